# Initial kernel scaffold; baseline (speedup 1.0000x reference)
import functools

import jax
import jax.numpy as jnp
from jax import lax
from jax.experimental import pallas as pl
from jax.experimental.pallas import tpu as pltpu
from jax.experimental.pallas import tpu_sc as plsc

_N_E = 8192
_E_DIM = 256
_BM = 256
_N_TOK = 8192


def _argmin_body(a_ref, b_ref, zf_ref, w_ref, idx_ref):
    zfb = zf_ref[...]
    w = w_ref[...]
    c = lax.dot_general(zfb, w, (((1,), (1,)), ((), ())),
                        preferred_element_type=jnp.float32)
    d = (a_ref[...] + b_ref[...]) - 2.0 * c
    m = jnp.min(d, axis=1, keepdims=True)
    ji = lax.broadcasted_iota(jnp.int32, d.shape, 1)
    idx = jnp.min(jnp.where(d == m, ji, jnp.int32(2147483647)),
                  axis=1, keepdims=True)
    idx_ref[...] = idx


def _argmin_call(a, b, zf, w):
    grid = (_N_TOK // _BM,)
    return pl.pallas_call(
        _argmin_body,
        grid=grid,
        in_specs=[
            pl.BlockSpec((_BM, 1), lambda i: (i, 0)),
            pl.BlockSpec((1, _N_E), lambda i: (0, 0)),
            pl.BlockSpec((_BM, _E_DIM), lambda i: (i, 0)),
            pl.BlockSpec((_N_E, _E_DIM), lambda i: (0, 0)),
        ],
        out_specs=pl.BlockSpec((_BM, 1), lambda i: (i, 0)),
        out_shape=jax.ShapeDtypeStruct((_N_TOK, 1), jnp.int32),
    )(a, b, zf, w)


_SC_INFO = plsc.get_sparse_core_info()
_NW = _SC_INFO.num_cores * _SC_INFO.num_subcores
_B_PER_W = _N_TOK // _NW


@functools.partial(
    pl.kernel,
    mesh=plsc.VectorSubcoreMesh(core_axis_name="c", subcore_axis_name="s"),
    out_type=jax.ShapeDtypeStruct((_N_TOK, _E_DIM), jnp.float32),
    scratch_types=[
        pltpu.VMEM((_B_PER_W,), jnp.int32),
        pltpu.VMEM((_B_PER_W, _E_DIM), jnp.float32),
        pltpu.SemaphoreType.DMA,
    ],
)
def _sc_gather(w_hbm, idx_hbm, out_hbm, idx_v, rows_v, sem):
    wid = lax.axis_index("s") * _SC_INFO.num_cores + lax.axis_index("c")
    base = wid * _B_PER_W
    pltpu.sync_copy(idx_hbm.at[pl.ds(base, _B_PER_W)], idx_v)
    pltpu.async_copy(w_hbm.at[idx_v], rows_v, sem).wait()
    pltpu.sync_copy(rows_v, out_hbm.at[pl.ds(base, _B_PER_W)])


def _ste_loss_body(zf_ref, zq_ref, ste_ref, sum_ref):
    zfb = zf_ref[...]
    zqb = zq_ref[...]
    ste_ref[...] = zfb + (zqb - zfb)
    sum_ref[0, 0] = jnp.sum((zqb - zfb) ** 2)


def _ste_loss_call(zf, zq):
    return pl.pallas_call(
        _ste_loss_body,
        out_specs=[
            pl.BlockSpec(memory_space=pltpu.VMEM),
            pl.BlockSpec(memory_space=pltpu.SMEM),
        ],
        out_shape=[
            jax.ShapeDtypeStruct((_N_TOK, _E_DIM), jnp.float32),
            jax.ShapeDtypeStruct((1, 1), jnp.float32),
        ],
    )(zf, zq)


def kernel(z, W):
    zp = jnp.transpose(z, (0, 2, 3, 1))
    zf = zp.reshape(-1, _E_DIM)
    a = jnp.sum(zf ** 2, axis=1, keepdims=True)
    b = jnp.sum(W ** 2, axis=1)[None, :]
    idx = _argmin_call(a, b, zf, W).reshape(_N_TOK)
    zq = _sc_gather(W, idx)
    ste, s = _ste_loss_call(zf, zq)
    m = s[0, 0] / jnp.float32(_N_TOK * _E_DIM)
    loss = m + 0.25 * m
    z_q = ste.reshape(zp.shape).transpose(0, 3, 1, 2)
    return (z_q, loss)


# speedup vs baseline: 1.1040x; 1.1040x over previous
import functools

import jax
import jax.numpy as jnp
from jax import lax
from jax.experimental import pallas as pl
from jax.experimental.pallas import tpu as pltpu
from jax.experimental.pallas import tpu_sc as plsc

_N_E = 8192
_E_DIM = 256
_BM = 256
_N_TOK = 8192


_WIN = 2048


def _argmin_body(a_ref, b_ref, zf_ref, w_ref, idx_ref):
    zfb = zf_ref[...]
    w = w_ref[...]
    c = lax.dot_general(zfb, w, (((1,), (1,)), ((), ())),
                        preferred_element_type=jnp.float32)
    d = (a_ref[...] + b_ref[...]) - 2.0 * c
    acc_v = jnp.full((_BM, 1), jnp.inf, jnp.float32)
    acc_i = jnp.zeros((_BM, 1), jnp.int32)
    for wnd in range(_N_E // _WIN):
        dw = d[:, wnd * _WIN:(wnd + 1) * _WIN]
        m = jnp.min(dw, axis=1, keepdims=True)
        ji = lax.broadcasted_iota(jnp.int32, dw.shape, 1) + jnp.int32(wnd * _WIN)
        iw = jnp.min(jnp.where(dw == m, ji, jnp.int32(2147483647)),
                     axis=1, keepdims=True)
        upd = m < acc_v
        acc_v = jnp.where(upd, m.astype(jnp.bfloat16).astype(jnp.float32), acc_v)
        acc_i = jnp.where(upd, iw, acc_i)
    idx_ref[...] = acc_i


def _argmin_call(a, b, zf, w):
    grid = (_N_TOK // _BM,)
    return pl.pallas_call(
        _argmin_body,
        grid=grid,
        in_specs=[
            pl.BlockSpec((_BM, 1), lambda i: (i, 0)),
            pl.BlockSpec((1, _N_E), lambda i: (0, 0)),
            pl.BlockSpec((_BM, _E_DIM), lambda i: (i, 0)),
            pl.BlockSpec((_N_E, _E_DIM), lambda i: (0, 0)),
        ],
        out_specs=pl.BlockSpec((_BM, 1), lambda i: (i, 0)),
        out_shape=jax.ShapeDtypeStruct((_N_TOK, 1), jnp.int32),
    )(a, b, zf, w)


_NC = 2
_NS = 16
_NW = _NC * _NS
_B_PER_W = _N_TOK // _NW


@functools.lru_cache(maxsize=None)
def _make_sc_gather():
    @functools.partial(
        pl.kernel,
        mesh=plsc.VectorSubcoreMesh(core_axis_name="c", subcore_axis_name="s"),
        out_type=jax.ShapeDtypeStruct((_N_TOK, _E_DIM), jnp.float32),
        scratch_types=[
            pltpu.VMEM((_B_PER_W,), jnp.int32),
            pltpu.VMEM((_B_PER_W, _E_DIM), jnp.float32),
            pltpu.SemaphoreType.DMA,
        ],
    )
    def _sc_gather(w_hbm, idx_hbm, out_hbm, idx_v, rows_v, sem):
        wid = lax.axis_index("s") * _NC + lax.axis_index("c")
        base = wid * _B_PER_W
        pltpu.sync_copy(idx_hbm.at[pl.ds(base, _B_PER_W)], idx_v)
        pltpu.async_copy(w_hbm.at[idx_v], rows_v, sem).wait()
        pltpu.sync_copy(rows_v, out_hbm.at[pl.ds(base, _B_PER_W)])

    return _sc_gather


def _ste_loss_body(zf_ref, zq_ref, ste_ref, sum_ref):
    zfb = zf_ref[...]
    zqb = zq_ref[...]
    ste_ref[...] = zfb + (zqb - zfb)
    sum_ref[0, 0] = jnp.sum((zqb - zfb) ** 2)


def _ste_loss_call(zf, zq):
    return pl.pallas_call(
        _ste_loss_body,
        out_specs=[
            pl.BlockSpec(memory_space=pltpu.VMEM),
            pl.BlockSpec(memory_space=pltpu.SMEM),
        ],
        out_shape=[
            jax.ShapeDtypeStruct((_N_TOK, _E_DIM), jnp.float32),
            jax.ShapeDtypeStruct((1, 1), jnp.float32),
        ],
    )(zf, zq)


def kernel(z, W):
    zp = jnp.transpose(z, (0, 2, 3, 1))
    zf = zp.reshape(-1, _E_DIM)
    a = jnp.sum(zf ** 2, axis=1, keepdims=True)
    b = jnp.sum(W ** 2, axis=1)[None, :]
    idx = _argmin_call(a, b, zf, W).reshape(_N_TOK)
    zq = _make_sc_gather()(W, idx)
    ste, s = _ste_loss_call(zf, zq)
    m = s[0, 0] / jnp.float32(_N_TOK * _E_DIM)
    loss = m + 0.25 * m
    z_q = ste.reshape(zp.shape).transpose(0, 3, 1, 2)
    return (z_q, loss)


# speedup vs baseline: 1.2334x; 1.1173x over previous
import functools

import jax
import jax.numpy as jnp
from jax import lax
from jax.experimental import pallas as pl
from jax.experimental.pallas import tpu as pltpu
from jax.experimental.pallas import tpu_sc as plsc

_N_E = 8192
_E_DIM = 256
_BM = 256
_N_TOK = 8192


_WIN = 2048


def _argmin_body(a_ref, b_ref, zf_ref, w2_ref, idx_ref):
    zfb = zf_ref[...]
    w2 = w2_ref[...]
    c2 = lax.dot_general(zfb, w2, (((1,), (1,)), ((), ())),
                         preferred_element_type=jnp.float32)
    d = (a_ref[...] + b_ref[...]) + c2
    acc_v = jnp.full((_BM, 1), jnp.inf, jnp.float32)
    acc_i = jnp.full((_BM, 1), 0.0, jnp.float32)
    for wnd in range(_N_E // _WIN):
        dw = d[:, wnd * _WIN:(wnd + 1) * _WIN]
        m = jnp.min(dw, axis=1, keepdims=True)
        ji = (lax.broadcasted_iota(jnp.int32, dw.shape, 1)
              + jnp.int32(wnd * _WIN)).astype(jnp.float32)
        iw = jnp.min(jnp.where(dw == m, ji, jnp.float32(jnp.inf)),
                     axis=1, keepdims=True)
        upd = m < acc_v
        acc_v = jnp.where(upd, m.astype(jnp.bfloat16).astype(jnp.float32), acc_v)
        acc_i = jnp.where(upd, iw, acc_i)
    idx_ref[...] = acc_i.astype(jnp.int32)


def _argmin_call(a, b, zf, w):
    grid = (_N_TOK // _BM,)
    return pl.pallas_call(
        _argmin_body,
        grid=grid,
        in_specs=[
            pl.BlockSpec((_BM, 1), lambda i: (i, 0)),
            pl.BlockSpec((1, _N_E), lambda i: (0, 0)),
            pl.BlockSpec((_BM, _E_DIM), lambda i: (i, 0)),
            pl.BlockSpec((_N_E, _E_DIM), lambda i: (0, 0)),
        ],
        out_specs=pl.BlockSpec((_BM, 1), lambda i: (i, 0)),
        out_shape=jax.ShapeDtypeStruct((_N_TOK, 1), jnp.int32),
    )(a, b, zf, w)


_NC = 2
_NS = 16
_NW = _NC * _NS
_B_PER_W = _N_TOK // _NW


@functools.lru_cache(maxsize=None)
def _make_sc_gather():
    @functools.partial(
        pl.kernel,
        mesh=plsc.VectorSubcoreMesh(core_axis_name="c", subcore_axis_name="s"),
        out_type=jax.ShapeDtypeStruct((_N_TOK, _E_DIM), jnp.float32),
        scratch_types=[
            pltpu.VMEM((_B_PER_W,), jnp.int32),
            pltpu.VMEM((_B_PER_W, _E_DIM), jnp.float32),
            pltpu.SemaphoreType.DMA,
        ],
    )
    def _sc_gather(w_hbm, idx_hbm, out_hbm, idx_v, rows_v, sem):
        wid = lax.axis_index("s") * _NC + lax.axis_index("c")
        base = wid * _B_PER_W
        pltpu.sync_copy(idx_hbm.at[pl.ds(base, _B_PER_W)], idx_v)
        pltpu.async_copy(w_hbm.at[idx_v], rows_v, sem).wait()
        pltpu.sync_copy(rows_v, out_hbm.at[pl.ds(base, _B_PER_W)])

    return _sc_gather


def _ste_loss_body(zf_ref, zq_ref, ste_ref, sum_ref):
    zfb = zf_ref[...]
    zqb = zq_ref[...]
    ste_ref[...] = zfb + (zqb - zfb)
    sum_ref[0, 0] = jnp.sum((zqb - zfb) ** 2)


def _ste_loss_call(zf, zq):
    return pl.pallas_call(
        _ste_loss_body,
        out_specs=[
            pl.BlockSpec(memory_space=pltpu.VMEM),
            pl.BlockSpec(memory_space=pltpu.SMEM),
        ],
        out_shape=[
            jax.ShapeDtypeStruct((_N_TOK, _E_DIM), jnp.float32),
            jax.ShapeDtypeStruct((1, 1), jnp.float32),
        ],
    )(zf, zq)


def kernel(z, W):
    zp = jnp.transpose(z, (0, 2, 3, 1))
    zf = zp.reshape(-1, _E_DIM)
    a = jnp.sum(zf ** 2, axis=1, keepdims=True)
    b = jnp.sum(W ** 2, axis=1)[None, :]
    idx = _argmin_call(a, b, zf, -2.0 * W).reshape(_N_TOK)
    zq = _make_sc_gather()(W, idx)
    ste, s = _ste_loss_call(zf, zq)
    m = s[0, 0] / jnp.float32(_N_TOK * _E_DIM)
    loss = m + 0.25 * m
    z_q = ste.reshape(zp.shape).transpose(0, 3, 1, 2)
    return (z_q, loss)
